# Initial kernel scaffold; baseline (speedup 1.0000x reference)
import jax
import jax.numpy as jnp
from jax.experimental import pallas as pl


def kernel(field, wx, wy, elements):
    raise NotImplementedError("write your pallas kernel here")


# speedup vs baseline: 5.7856x; 5.7856x over previous
import functools

import jax
import jax.numpy as jnp
from jax import lax
from jax.experimental import pallas as pl
from jax.experimental.pallas import tpu as pltpu
from jax.experimental.pallas import tpu_sc as plsc

_LANES = 16
_NUM_WORKERS = 32


def _pick_chunk(per_w: int, budget_words: int) -> int:
    best = _LANES
    for groups in range(1, per_w // _LANES + 1):
        c = groups * _LANES
        if per_w % c == 0 and 11 * c <= budget_words:
            best = c
    return best


@functools.lru_cache(maxsize=None)
def _build_impl(E: int, V_pad: int):
    per_w = ((E + _NUM_WORKERS - 1) // _NUM_WORKERS + _LANES - 1) // _LANES * _LANES
    budget = 131000 - V_pad
    chunk = _pick_chunk(per_w, budget)
    nsub = per_w // chunk

    mesh = plsc.VectorSubcoreMesh(core_axis_name="c", subcore_axis_name="s")

    @functools.partial(
        pl.kernel,
        out_type=jax.ShapeDtypeStruct((2 * E,), jnp.float32),
        mesh=mesh,
        compiler_params=pltpu.CompilerParams(needs_layout_passes=False),
        scratch_types=[
            pltpu.VMEM((V_pad,), jnp.float32),
            pltpu.VMEM((3 * chunk,), jnp.int32),
            pltpu.VMEM((3 * chunk,), jnp.float32),
            pltpu.VMEM((3 * chunk,), jnp.float32),
            pltpu.VMEM((2 * chunk,), jnp.float32),
        ],
    )
    def _impl(field_hbm, el_hbm, wx_hbm, wy_hbm, out_hbm,
              field_v, el_v, wx_v, wy_v, out_v):
        cid = lax.axis_index("c")
        sid = lax.axis_index("s")
        wid = (sid * 2 + cid).astype(jnp.int32)
        base_w = jnp.minimum(wid * per_w, E - per_w)

        pltpu.sync_copy(field_hbm, field_v)

        iota = lax.iota(jnp.int32, _LANES)

        def sub_body(s, _):
            sc_base = base_w + s * chunk
            pltpu.sync_copy(el_hbm.at[pl.ds(3 * sc_base, 3 * chunk)], el_v)
            pltpu.sync_copy(wx_hbm.at[pl.ds(3 * sc_base, 3 * chunk)], wx_v)
            pltpu.sync_copy(wy_hbm.at[pl.ds(3 * sc_base, 3 * chunk)], wy_v)

            def step(j, _):
                rows = j * _LANES + iota
                r3 = rows * 3
                e0 = plsc.load_gather(el_v, [r3])
                e1 = plsc.load_gather(el_v, [r3 + 1])
                e2 = plsc.load_gather(el_v, [r3 + 2])
                f0 = plsc.load_gather(field_v, [e0])
                f1 = plsc.load_gather(field_v, [e1])
                f2 = plsc.load_gather(field_v, [e2])
                wx0 = plsc.load_gather(wx_v, [r3])
                wx1 = plsc.load_gather(wx_v, [r3 + 1])
                wx2 = plsc.load_gather(wx_v, [r3 + 2])
                wy0 = plsc.load_gather(wy_v, [r3])
                wy1 = plsc.load_gather(wy_v, [r3 + 1])
                wy2 = plsc.load_gather(wy_v, [r3 + 2])
                gx = wx0 * f0 + wx1 * f1 + wx2 * f2
                gy = wy0 * f0 + wy1 * f1 + wy2 * f2
                r2 = rows * 2
                plsc.store_scatter(out_v, [r2], gx)
                plsc.store_scatter(out_v, [r2 + 1], gy)
                return 0

            lax.fori_loop(0, chunk // _LANES, step, 0)
            pltpu.sync_copy(out_v, out_hbm.at[pl.ds(2 * sc_base, 2 * chunk)])
            return 0

        lax.fori_loop(0, nsub, sub_body, 0)

    return _impl


def kernel(field, wx, wy, elements):
    V = field.shape[0]
    E = elements.shape[0]
    V_pad = (V + _LANES - 1) // _LANES * _LANES
    field_p = jnp.pad(field, (0, V_pad - V)) if V_pad != V else field
    el = elements.astype(jnp.int32).reshape(-1)
    impl = _build_impl(E, V_pad)
    out_flat = impl(field_p, el,
                    wx.astype(jnp.float32).reshape(-1),
                    wy.astype(jnp.float32).reshape(-1))
    return out_flat.reshape(E, 2)


# speedup vs baseline: 41.8390x; 7.2316x over previous
import functools

import jax
import jax.numpy as jnp
from jax import lax
from jax.experimental import pallas as pl
from jax.experimental.pallas import tpu as pltpu
from jax.experimental.pallas import tpu_sc as plsc

_LANES = 16
_NUM_WORKERS = 32


@functools.lru_cache(maxsize=None)
def _build_impl(E: int, V_pad: int):
    budget = 131000 - V_pad
    chunk = (budget // 11) // 128 * 128
    nsub = 1
    while nsub * chunk * _NUM_WORKERS < E + _NUM_WORKERS * _LANES:
        nsub += 1
    per_w = nsub * chunk
    assert per_w <= E
    step = -((E - per_w) // -(_NUM_WORKERS - 1))
    step = -(step // -_LANES) * _LANES

    mesh = plsc.VectorSubcoreMesh(core_axis_name="c", subcore_axis_name="s")

    @functools.partial(
        pl.kernel,
        out_type=jax.ShapeDtypeStruct((2 * E,), jnp.float32),
        mesh=mesh,
        compiler_params=pltpu.CompilerParams(needs_layout_passes=False),
        scratch_types=[
            pltpu.VMEM((V_pad,), jnp.float32),
            pltpu.VMEM((3 * chunk,), jnp.int32),
            pltpu.VMEM((3 * chunk,), jnp.float32),
            pltpu.VMEM((3 * chunk,), jnp.float32),
            pltpu.VMEM((chunk,), jnp.float32),
            pltpu.VMEM((chunk,), jnp.float32),
        ],
    )
    def _impl(field_hbm, el_hbm, wx_hbm, wy_hbm, out_hbm,
              field_v, el_v, wx_v, wy_v, gx_v, gy_v):
        cid = lax.axis_index("c")
        sid = lax.axis_index("s")
        wid = (sid * 2 + cid).astype(jnp.int32)
        base_w = jnp.minimum(wid * step, E - per_w)

        pltpu.sync_copy(field_hbm, field_v)

        def sub_body(s, _):
            sc_base = base_w + s * chunk
            for k in range(3):
                pltpu.sync_copy(el_hbm.at[pl.ds(k * E + sc_base, chunk)],
                                el_v.at[pl.ds(k * chunk, chunk)])
                pltpu.sync_copy(wx_hbm.at[pl.ds(k * E + sc_base, chunk)],
                                wx_v.at[pl.ds(k * chunk, chunk)])
                pltpu.sync_copy(wy_hbm.at[pl.ds(k * E + sc_base, chunk)],
                                wy_v.at[pl.ds(k * chunk, chunk)])

            def loop_step(j, _):
                o = j * _LANES
                e0 = el_v[pl.ds(o, _LANES)]
                e1 = el_v[pl.ds(chunk + o, _LANES)]
                e2 = el_v[pl.ds(2 * chunk + o, _LANES)]
                f0 = plsc.load_gather(field_v, [e0])
                f1 = plsc.load_gather(field_v, [e1])
                f2 = plsc.load_gather(field_v, [e2])
                gx = (wx_v[pl.ds(o, _LANES)] * f0
                      + wx_v[pl.ds(chunk + o, _LANES)] * f1
                      + wx_v[pl.ds(2 * chunk + o, _LANES)] * f2)
                gy = (wy_v[pl.ds(o, _LANES)] * f0
                      + wy_v[pl.ds(chunk + o, _LANES)] * f1
                      + wy_v[pl.ds(2 * chunk + o, _LANES)] * f2)
                gx_v[pl.ds(o, _LANES)] = gx
                gy_v[pl.ds(o, _LANES)] = gy
                return 0

            lax.fori_loop(0, chunk // _LANES, loop_step, 0)
            pltpu.sync_copy(gx_v, out_hbm.at[pl.ds(sc_base, chunk)])
            pltpu.sync_copy(gy_v, out_hbm.at[pl.ds(E + sc_base, chunk)])
            return 0

        lax.fori_loop(0, nsub, sub_body, 0)

    return _impl


def kernel(field, wx, wy, elements):
    V = field.shape[0]
    E = elements.shape[0]
    V_pad = (V + 127) // 128 * 128
    field_p = jnp.pad(field, (0, V_pad - V)) if V_pad != V else field
    elT = elements.astype(jnp.int32).T.reshape(-1)
    wxT = wx.T.reshape(-1)
    wyT = wy.T.reshape(-1)
    impl = _build_impl(E, V_pad)
    out = impl(field_p, elT, wxT, wyT)
    return jnp.stack([out[:E], out[E:]], axis=1)


# speedup vs baseline: 60.0901x; 1.4362x over previous
import functools

import jax
import jax.numpy as jnp
from jax import lax
from jax.experimental import pallas as pl
from jax.experimental.pallas import tpu as pltpu
from jax.experimental.pallas import tpu_sc as plsc

_LANES = 16
_NUM_WORKERS = 32
_CHUNK = 1280
_NSUB = 5


@functools.lru_cache(maxsize=None)
def _build_impl(E: int, V_pad: int):
    E_pad = -(E // -128) * 128
    per_w = _NSUB * _CHUNK
    last_base = E_pad - per_w
    step = -(last_base // -(128 * (_NUM_WORKERS - 1))) * 128
    assert (_NUM_WORKERS - 2) * step + per_w >= last_base, "coverage gap"

    mesh = plsc.VectorSubcoreMesh(core_axis_name="c", subcore_axis_name="s")

    in_block = lambda dt: pltpu.VMEM((3 * _CHUNK,), dt)

    @functools.partial(
        pl.kernel,
        out_type=jax.ShapeDtypeStruct((2 * E_pad,), jnp.float32),
        mesh=mesh,
        compiler_params=pltpu.CompilerParams(needs_layout_passes=False),
        scratch_types=[
            pltpu.VMEM((V_pad,), jnp.float32),
            [in_block(jnp.int32) for _ in range(2)],
            [in_block(jnp.float32) for _ in range(2)],
            [in_block(jnp.float32) for _ in range(2)],
            [pltpu.VMEM((2 * _CHUNK,), jnp.float32) for _ in range(2)],
            pltpu.SemaphoreType.DMA,
            [pltpu.SemaphoreType.DMA for _ in range(2)],
            [pltpu.SemaphoreType.DMA for _ in range(2)],
        ],
    )
    def _impl(field_hbm, el_hbm, wx_hbm, wy_hbm, out_hbm,
              field_v, el_v, wx_v, wy_v, out_v, sem_f, sem_in, sem_out):
        cid = lax.axis_index("c")
        sid = lax.axis_index("s")
        wid = (sid * 2 + cid).astype(jnp.int32)
        base_w = jnp.minimum(wid * step, last_base)

        h_field = pltpu.async_copy(field_hbm, field_v, sem_f)

        def issue_in(s, b):
            sc_base = base_w + s * _CHUNK
            hs = []
            for arr_hbm, arr_v in ((el_hbm, el_v), (wx_hbm, wx_v), (wy_hbm, wy_v)):
                for k in range(3):
                    hs.append(pltpu.async_copy(
                        arr_hbm.at[pl.ds(k * E + sc_base, _CHUNK)],
                        arr_v[b].at[pl.ds(k * _CHUNK, _CHUNK)],
                        sem_in[b]))
            return hs

        pend_in = {0: issue_in(0, 0)}
        pend_out = {}

        for s in range(_NSUB):
            b = s % 2
            if s + 1 < _NSUB:
                pend_in[s + 1] = issue_in(s + 1, (s + 1) % 2)
            for h in pend_in.pop(s):
                h.wait()
            if s == 0:
                h_field.wait()
            if s >= 2:
                pend_out.pop(s - 2).wait()

            def body(g, _, b=b):
                for u in range(8):
                    o = g * 128 + u * _LANES
                    e0 = el_v[b][pl.ds(o, _LANES)]
                    e1 = el_v[b][pl.ds(_CHUNK + o, _LANES)]
                    e2 = el_v[b][pl.ds(2 * _CHUNK + o, _LANES)]
                    f0 = plsc.load_gather(field_v, [e0])
                    f1 = plsc.load_gather(field_v, [e1])
                    f2 = plsc.load_gather(field_v, [e2])
                    gx = (wx_v[b][pl.ds(o, _LANES)] * f0
                          + wx_v[b][pl.ds(_CHUNK + o, _LANES)] * f1
                          + wx_v[b][pl.ds(2 * _CHUNK + o, _LANES)] * f2)
                    gy = (wy_v[b][pl.ds(o, _LANES)] * f0
                          + wy_v[b][pl.ds(_CHUNK + o, _LANES)] * f1
                          + wy_v[b][pl.ds(2 * _CHUNK + o, _LANES)] * f2)
                    d = g * 256 + u * _LANES
                    out_v[b][pl.ds(d, _LANES)] = gx
                    out_v[b][pl.ds(d + 128, _LANES)] = gy
                return 0

            lax.fori_loop(0, _CHUNK // 128, body, 0)

            sc_base = base_w + s * _CHUNK
            pend_out[s] = pltpu.async_copy(
                out_v[b], out_hbm.at[pl.ds(2 * sc_base, 2 * _CHUNK)], sem_out[b])

        for s, h in sorted(pend_out.items()):
            h.wait()

    return _impl


def kernel(field, wx, wy, elements):
    V = field.shape[0]
    E = elements.shape[0]
    E_pad = -(E // -128) * 128
    V_pad = -(V // -128) * 128
    field_p = jnp.pad(field, (0, V_pad - V)) if V_pad != V else field
    tail = E_pad - E
    elT = jnp.pad(elements.astype(jnp.int32).T.reshape(-1), (0, tail))
    wxT = jnp.pad(wx.T.reshape(-1), (0, tail))
    wyT = jnp.pad(wy.T.reshape(-1), (0, tail))
    impl = _build_impl(E, V_pad)
    out = impl(field_p, elT, wxT, wyT)
    return out.reshape(E_pad // 128, 2, 128).transpose(0, 2, 1).reshape(E_pad, 2)[:E]


# speedup vs baseline: 64.4628x; 1.0728x over previous
import functools

import jax
import jax.numpy as jnp
from jax import lax
from jax.experimental import pallas as pl
from jax.experimental.pallas import tpu as pltpu
from jax.experimental.pallas import tpu_sc as plsc

_LANES = 16
_NUM_WORKERS = 32
_CHUNK = 1280
_NSUB = 5


@functools.lru_cache(maxsize=None)
def _build_impl(E: int, V: int):
    V_pad = -(V // -128) * 128
    E_pad = -(E // -128) * 128
    per_w = _NSUB * _CHUNK
    last_base = E_pad - per_w
    step = -(last_base // -(128 * (_NUM_WORKERS - 1))) * 128
    assert (_NUM_WORKERS - 2) * step + per_w >= last_base, "coverage gap"

    mesh = plsc.VectorSubcoreMesh(core_axis_name="c", subcore_axis_name="s")

    in_block = lambda dt: pltpu.VMEM((3 * _CHUNK,), dt)

    @functools.partial(
        pl.kernel,
        out_type=jax.ShapeDtypeStruct((2 * E_pad,), jnp.float32),
        mesh=mesh,
        compiler_params=pltpu.CompilerParams(needs_layout_passes=False),
        scratch_types=[
            pltpu.VMEM((V_pad,), jnp.float32),
            [in_block(jnp.int32) for _ in range(2)],
            [in_block(jnp.float32) for _ in range(2)],
            [in_block(jnp.float32) for _ in range(2)],
            [pltpu.VMEM((2 * _CHUNK,), jnp.float32) for _ in range(2)],
            pltpu.SemaphoreType.DMA,
            [pltpu.SemaphoreType.DMA for _ in range(2)],
            [pltpu.SemaphoreType.DMA for _ in range(2)],
        ],
    )
    def _impl(field_hbm, el_hbm, wx_hbm, wy_hbm, out_hbm,
              field_v, el_v, wx_v, wy_v, out_v, sem_f, sem_in, sem_out):
        cid = lax.axis_index("c")
        sid = lax.axis_index("s")
        wid = (sid * 2 + cid).astype(jnp.int32)
        base_w = jnp.minimum(wid * step, last_base)

        h_field = pltpu.async_copy(field_hbm, field_v.at[pl.ds(0, V)], sem_f)

        def issue_in(s, b):
            sc_base = base_w + s * _CHUNK
            hs = []
            for arr_hbm, arr_v in ((el_hbm, el_v), (wx_hbm, wx_v), (wy_hbm, wy_v)):
                for k in range(3):
                    hs.append(pltpu.async_copy(
                        arr_hbm.at[pl.ds(k * E + sc_base, _CHUNK)],
                        arr_v[b].at[pl.ds(k * _CHUNK, _CHUNK)],
                        sem_in[b]))
            return hs

        pend_in = {0: issue_in(0, 0)}
        pend_out = {}

        for s in range(_NSUB):
            b = s % 2
            if s + 1 < _NSUB:
                pend_in[s + 1] = issue_in(s + 1, (s + 1) % 2)
            for h in pend_in.pop(s):
                h.wait()
            if s == 0:
                h_field.wait()
            if s >= 2:
                pend_out.pop(s - 2).wait()

            def body(g, _, b=b):
                for u in range(8):
                    o = g * 128 + u * _LANES
                    hi = jnp.full((_LANES,), V - 1, jnp.int32)
                    lo = jnp.zeros((_LANES,), jnp.int32)
                    clamp = lambda e: jnp.minimum(jnp.maximum(e, lo), hi)
                    e0 = clamp(el_v[b][pl.ds(o, _LANES)])
                    e1 = clamp(el_v[b][pl.ds(_CHUNK + o, _LANES)])
                    e2 = clamp(el_v[b][pl.ds(2 * _CHUNK + o, _LANES)])
                    f0 = plsc.load_gather(field_v, [e0])
                    f1 = plsc.load_gather(field_v, [e1])
                    f2 = plsc.load_gather(field_v, [e2])
                    gx = (wx_v[b][pl.ds(o, _LANES)] * f0
                          + wx_v[b][pl.ds(_CHUNK + o, _LANES)] * f1
                          + wx_v[b][pl.ds(2 * _CHUNK + o, _LANES)] * f2)
                    gy = (wy_v[b][pl.ds(o, _LANES)] * f0
                          + wy_v[b][pl.ds(_CHUNK + o, _LANES)] * f1
                          + wy_v[b][pl.ds(2 * _CHUNK + o, _LANES)] * f2)
                    d = g * 256 + u * _LANES
                    out_v[b][pl.ds(d, _LANES)] = gx
                    out_v[b][pl.ds(d + 128, _LANES)] = gy
                return 0

            lax.fori_loop(0, _CHUNK // 128, body, 0)

            sc_base = base_w + s * _CHUNK
            pend_out[s] = pltpu.async_copy(
                out_v[b], out_hbm.at[pl.ds(2 * sc_base, 2 * _CHUNK)], sem_out[b])

        for s, h in sorted(pend_out.items()):
            h.wait()

    return _impl


def kernel(field, wx, wy, elements):
    V = field.shape[0]
    E = elements.shape[0]
    E_pad = -(E // -128) * 128
    elT = elements.astype(jnp.int32).T.reshape(-1)
    wxT = wx.T.reshape(-1)
    wyT = wy.T.reshape(-1)
    impl = _build_impl(E, V)
    out = impl(field, elT, wxT, wyT)
    return out.reshape(E_pad // 128, 2, 128).transpose(0, 2, 1).reshape(E_pad, 2)[:E]


# speedup vs baseline: 73.2393x; 1.1361x over previous
import functools

import jax
import jax.numpy as jnp
from jax import lax
from jax.experimental import pallas as pl
from jax.experimental.pallas import tpu as pltpu
from jax.experimental.pallas import tpu_sc as plsc

_LANES = 16
_NUM_WORKERS = 32
_NSC = 16
_CHUNK = 3200
_NSUB = 2


@functools.lru_cache(maxsize=None)
def _build_impl(E: int, V: int):
    V_pad = -(V // -128) * 128
    E_pad = -(E // -128) * 128
    per_w = _NSUB * _CHUNK
    last_base = E_pad - per_w
    step = -(last_base // -(128 * (_NUM_WORKERS - 1))) * 128
    assert step <= per_w and (_NUM_WORKERS - 2) * step + per_w >= last_base
    stripe = V_pad // _NSC
    assert stripe % 8 == 0

    mesh = plsc.VectorSubcoreMesh(core_axis_name="c", subcore_axis_name="s")

    in_block = lambda dt: pltpu.VMEM((3 * _CHUNK,), dt)

    @functools.partial(
        pl.kernel,
        out_type=jax.ShapeDtypeStruct((2 * E_pad,), jnp.float32),
        mesh=mesh,
        compiler_params=pltpu.CompilerParams(needs_layout_passes=False),
        scratch_types=[
            pltpu.VMEM_SHARED((V_pad,), jnp.float32),
            [in_block(jnp.int32) for _ in range(2)],
            [in_block(jnp.float32) for _ in range(2)],
            [in_block(jnp.float32) for _ in range(2)],
            [in_block(jnp.float32) for _ in range(2)],
            [pltpu.VMEM((2 * _CHUNK,), jnp.float32) for _ in range(2)],
            [pltpu.SemaphoreType.DMA for _ in range(2)],
            [pltpu.SemaphoreType.DMA for _ in range(2)],
            [pltpu.SemaphoreType.DMA for _ in range(2)],
        ],
    )
    def _impl(field_hbm, el_hbm, wx_hbm, wy_hbm, out_hbm,
              field_sh, el_v, wx_v, wy_v, f_v, out_v, sem_in, sem_fg, sem_out):
        cid = lax.axis_index("c")
        sid = lax.axis_index("s")
        wid = (sid * 2 + cid).astype(jnp.int32)
        base_w = jnp.minimum(wid * step, last_base)

        def issue_in(s, b):
            sc_base = base_w + s * _CHUNK
            hs = []
            for arr_hbm, arr_v in ((el_hbm, el_v), (wx_hbm, wx_v), (wy_hbm, wy_v)):
                for k in range(3):
                    hs.append(pltpu.async_copy(
                        arr_hbm.at[pl.ds(k * E + sc_base, _CHUNK)],
                        arr_v[b].at[pl.ds(k * _CHUNK, _CHUNK)],
                        sem_in[b]))
            return hs

        pend_in = {0: issue_in(0, 0), 1: issue_in(1, 1)}

        s_lo = sid * stripe
        pltpu.sync_copy(field_hbm.at[pl.ds(s_lo, stripe)],
                        f_v[0].at[pl.ds(0, stripe)])
        pltpu.sync_copy(f_v[0].at[pl.ds(0, stripe)],
                        field_sh.at[pl.ds(s_lo, stripe)])
        plsc.subcore_barrier()

        def fixup_tail(s, b):
            sc_base = base_w + s * _CHUNK
            @pl.when(sc_base + _CHUNK > E)
            def _():
                hi = jnp.full((_LANES,), V - 1, jnp.int32)
                lo = jnp.zeros((_LANES,), jnp.int32)
                for u in range(8):
                    o = 3 * _CHUNK - 128 + u * _LANES
                    el_v[b][pl.ds(o, _LANES)] = jnp.minimum(
                        jnp.maximum(el_v[b][pl.ds(o, _LANES)], lo), hi)

        def issue_gather(s, b):
            hs = []
            for k in range(3):
                hs.append(pltpu.async_copy(
                    field_sh.at[el_v[b].at[pl.ds(k * _CHUNK, _CHUNK)]],
                    f_v[b].at[pl.ds(k * _CHUNK, _CHUNK)],
                    sem_fg[b]))
            return hs

        for h in pend_in.pop(0):
            h.wait()
        fixup_tail(0, 0)
        pend_fg = {0: issue_gather(0, 0)}
        pend_out = {}

        for s in range(_NSUB):
            b = s % 2
            if s + 1 < _NSUB:
                nb = (s + 1) % 2
                for h in pend_in.pop(s + 1):
                    h.wait()
                fixup_tail(s + 1, nb)
                pend_fg[s + 1] = issue_gather(s + 1, nb)
            for h in pend_fg.pop(s):
                h.wait()
            if s >= 2:
                pend_out.pop(s - 2).wait()

            def body(g, _, b=b):
                for u in range(8):
                    o = g * 128 + u * _LANES
                    f0 = f_v[b][pl.ds(o, _LANES)]
                    f1 = f_v[b][pl.ds(_CHUNK + o, _LANES)]
                    f2 = f_v[b][pl.ds(2 * _CHUNK + o, _LANES)]
                    gx = (wx_v[b][pl.ds(o, _LANES)] * f0
                          + wx_v[b][pl.ds(_CHUNK + o, _LANES)] * f1
                          + wx_v[b][pl.ds(2 * _CHUNK + o, _LANES)] * f2)
                    gy = (wy_v[b][pl.ds(o, _LANES)] * f0
                          + wy_v[b][pl.ds(_CHUNK + o, _LANES)] * f1
                          + wy_v[b][pl.ds(2 * _CHUNK + o, _LANES)] * f2)
                    d = g * 256 + u * _LANES
                    out_v[b][pl.ds(d, _LANES)] = gx
                    out_v[b][pl.ds(d + 128, _LANES)] = gy
                return 0

            lax.fori_loop(0, _CHUNK // 128, body, 0)

            sc_base = base_w + s * _CHUNK
            pend_out[s] = pltpu.async_copy(
                out_v[b], out_hbm.at[pl.ds(2 * sc_base, 2 * _CHUNK)], sem_out[b])

        for s, h in sorted(pend_out.items()):
            h.wait()

    return _impl


def kernel(field, wx, wy, elements):
    V = field.shape[0]
    E = elements.shape[0]
    E_pad = -(E // -128) * 128
    elT = elements.astype(jnp.int32).T.reshape(-1)
    wxT = wx.T.reshape(-1)
    wyT = wy.T.reshape(-1)
    impl = _build_impl(E, V)
    out = impl(field, elT, wxT, wyT)
    return out.reshape(E_pad // 128, 2, 128).transpose(0, 2, 1).reshape(E_pad, 2)[:E]
